# baseline (speedup 1.0000x reference)
import functools

import jax
import jax.numpy as jnp
from jax import lax
from jax.experimental import pallas as pl
from jax.experimental.pallas import tpu as pltpu
from jax.experimental.pallas import tpu_sc as plsc

VOCAB = 1000
B = 4096
S = 20
CB = 4
LANES = 16

_info = plsc.get_sparse_core_info()
NC, NS = _info.num_cores, _info.num_subcores
NW = NC * NS
BPW = B // NW
NCHUNK = BPW // CB
IDX_PER_CHUNK = CB * S
NVEC = IDX_PER_CHUNK // LANES


def _sc_kernel(x_hbm, zeros_hbm, out_hbm, idxbuf, zbuf, sem):
    wid = lax.axis_index("s") * NC + lax.axis_index("c")
    pltpu.sync_copy(zeros_hbm, zbuf)

    def chunk(c, _):
        b0 = wid * BPW + c * CB
        pltpu.sync_copy(x_hbm.at[pl.ds(b0 * S, IDX_PER_CHUNK)], idxbuf)

        def scatter(val):
            for k in range(NVEC):
                p = lax.iota(jnp.int32, LANES) + (k * LANES)
                b_loc = lax.div(p, S)
                s_loc = lax.rem(p, S)
                v = idxbuf[pl.ds(k * LANES, LANES)]
                plsc.store_scatter(
                    zbuf, [b_loc, s_loc, v],
                    jnp.full((LANES,), val, jnp.float32),
                )

        scatter(1.0)
        pltpu.sync_copy(zbuf, out_hbm.at[pl.ds(b0, CB)])
        scatter(0.0)
        return 0

    lax.fori_loop(0, NCHUNK, chunk, 0)


def kernel(x):
    xi = x.astype(jnp.int32).reshape(B * S)
    zeros = jnp.zeros((CB, S, VOCAB), jnp.float32)
    mesh = plsc.VectorSubcoreMesh(core_axis_name="c", subcore_axis_name="s")
    k = functools.partial(
        pl.kernel,
        out_type=jax.ShapeDtypeStruct((B, S, VOCAB), jnp.float32),
        mesh=mesh,
        compiler_params=pltpu.CompilerParams(use_tc_tiling_on_sc=False, needs_layout_passes=False),
        scratch_types=[
            pltpu.VMEM((IDX_PER_CHUNK,), jnp.int32),
            pltpu.VMEM((CB, S, VOCAB), jnp.float32),
            pltpu.SemaphoreType.DMA,
        ],
    )(_sc_kernel)
    return k(xi, zeros)
